# baseline (speedup 1.0000x reference)
import functools

import jax
import jax.numpy as jnp
from jax import lax
from jax.experimental import pallas as pl
from jax.experimental.pallas import tpu as pltpu
from jax.experimental.pallas import tpu_sc as plsc


def _gather_kernel(B, V, D, NC, NS, CH):
  NW = NC * NS
  b_per_w = B // NW
  n_chunks = b_per_w // CH
  mesh = plsc.VectorSubcoreMesh(core_axis_name="c", subcore_axis_name="s")

  @functools.partial(
      pl.kernel,
      mesh=mesh,
      out_type=jax.ShapeDtypeStruct((B, D), jnp.float32),
      scratch_types=[
          pltpu.VMEM((n_chunks, CH), jnp.int32),
          pltpu.VMEM((b_per_w, D), jnp.float32),
          pltpu.SemaphoreType.DMA,
      ],
      compiler_params=pltpu.CompilerParams(use_tc_tiling_on_sc=False),
  )
  def k(idx_hbm, table_hbm, out_hbm, idx_v, rows_v, sem):
    wid = lax.axis_index("s") * NC + lax.axis_index("c")
    pltpu.sync_copy(idx_hbm.at[wid], idx_v)
    copies = []
    for j in range(n_chunks):
      copies.append(
          pltpu.async_copy(
              table_hbm.at[idx_v.at[j]], rows_v.at[pl.ds(j * CH, CH)], sem
          )
      )
    for c in copies:
      c.wait()
    pltpu.sync_copy(rows_v, out_hbm.at[pl.ds(wid * b_per_w, b_per_w)])

  return k


def kernel(input, table):
  B = input.shape[0]
  V, D = table.shape
  NC, NS, CH = 2, 16, 128
  NW = NC * NS
  assert B % (NW * CH) == 0
  idx3 = input.reshape(NW, B // (NW * CH), CH)
  out = _gather_kernel(B, V, D, NC, NS, CH)(idx3, table)
  return out.reshape((1, -1))
